# baseline (speedup 1.0000x reference)
import functools

import jax
import jax.numpy as jnp
import numpy as np
from jax import lax
from jax.experimental import pallas as pl
from jax.experimental.pallas import tpu as pltpu
from jax.experimental.pallas import tpu_sc as plsc

HEAD = 16
KD = 64
DM = 1024
ROUND = 4
BUCKETS = 64
NEG = float(np.finfo(np.float32).max)

SC_CORES = 2
SC_SUBCORES = 16
SC_WORKERS = SC_CORES * SC_SUBCORES


def _matmul_body(a_ref, w_ref, b_ref, o_ref):
    a = a_ref[...]
    w = w_ref[...]
    o_ref[...] = (
        jnp.dot(a, w, preferred_element_type=jnp.float32) + b_ref[...]
    )


def k_matmul(a, w, b, bm=512, bn=512):
    m, k = a.shape
    k2, n = w.shape
    assert k == k2 and b.shape == (n,)
    return pl.pallas_call(
        _matmul_body,
        grid=(m // bm, n // bn),
        in_specs=[
            pl.BlockSpec((bm, k), lambda i, j: (i, 0)),
            pl.BlockSpec((k, bn), lambda i, j: (0, j)),
            pl.BlockSpec((1, bn), lambda i, j: (0, j)),
        ],
        out_specs=pl.BlockSpec((bm, bn), lambda i, j: (i, j)),
        out_shape=jax.ShapeDtypeStruct((m, n), jnp.float32),
    )(a, w, b.reshape(1, n))


def _hash_body(qk_ref, hv_ref, o_ref):
    qk = qk_ref[0]
    hv = hv_ref[0]
    rot = jnp.dot(qk, hv, preferred_element_type=jnp.float32)
    s = rot.shape[0]
    cols = []
    half = BUCKETS // 2
    iota = lax.broadcasted_iota(jnp.int32, (s, half), 1)
    for r in range(ROUND):
        xr = rot[:, r * half:(r + 1) * half]
        m = jnp.max(jnp.abs(xr), axis=1, keepdims=True)
        negidx = jnp.min(jnp.where(xr == -m, iota, half), axis=1,
                         keepdims=True)
        posidx = jnp.min(jnp.where(xr == m, iota, half), axis=1,
                         keepdims=True)
        cols.append(jnp.where(negidx < half, negidx, posidx + half))
    o_ref[0] = jnp.concatenate(cols, axis=1)


def k_hash(qk_h, hv):
    bh, s, _ = qk_h.shape
    return pl.pallas_call(
        _hash_body,
        grid=(bh,),
        in_specs=[
            pl.BlockSpec((1, s, KD), lambda i: (i, 0, 0)),
            pl.BlockSpec((1, KD, ROUND * (BUCKETS // 2)),
                         lambda i: (i % HEAD, 0, 0)),
        ],
        out_specs=pl.BlockSpec((1, s, ROUND), lambda i: (i, 0, 0)),
        out_shape=jax.ShapeDtypeStruct((bh, s, ROUND), jnp.int32),
    )(qk_h, hv)


def _rank_body(bk_ref, posl_ref, posg_ref, *, s):
    kk = bk_ref[0]
    sp = kk.shape[0]
    lane_lt = (lax.broadcasted_iota(jnp.int32, (128, 128), 0)
               < lax.broadcasted_iota(jnp.int32, (128, 128), 1)
               ).astype(jnp.float32)
    row_lt = (lax.broadcasted_iota(jnp.int32, (sp, sp), 0)
              > lax.broadcasted_iota(jnp.int32, (sp, sp), 1)
              ).astype(jnp.float32)
    pos = jnp.zeros((sp, 128), jnp.float32)
    acc = jnp.float32(0.0)
    for v in range(BUCKETS):
        eq = (kk == v).astype(jnp.float32)
        lane_prefix = jnp.dot(eq, lane_lt, preferred_element_type=jnp.float32)
        rowsum = jnp.sum(eq, axis=1, keepdims=True)
        rowsum_b = jnp.broadcast_to(rowsum, (sp, 128))
        rowexcl = jnp.dot(row_lt, rowsum_b,
                          preferred_element_type=jnp.float32)
        total = jnp.sum(eq)
        pos = pos + eq * (acc + rowexcl + lane_prefix)
        acc = acc + total
    posl = pos.astype(jnp.int32)
    posl_ref[0] = posl
    posg_ref[0] = posl + pl.program_id(0) * s


def k_rank(bk):
    bhr, sp, _ = bk.shape
    s = sp * 128
    return pl.pallas_call(
        functools.partial(_rank_body, s=s),
        grid=(bhr,),
        in_specs=[pl.BlockSpec((1, sp, 128), lambda i: (i, 0, 0))],
        out_specs=[
            pl.BlockSpec((1, sp, 128), lambda i: (i, 0, 0)),
            pl.BlockSpec((1, sp, 128), lambda i: (i, 0, 0)),
        ],
        out_shape=[
            jax.ShapeDtypeStruct((bhr, sp, 128), jnp.int32),
            jax.ShapeDtypeStruct((bhr, sp, 128), jnp.int32),
        ],
    )(bk)


def _sc_sort_gather(posl, bkp, qkvf, *, bhr, s):
    tasks_per_w = bhr // SC_WORKERS
    ch = 128
    nch = s // ch
    mesh = plsc.VectorSubcoreMesh(core_axis_name="c", subcore_axis_name="s")

    @functools.partial(
        pl.kernel,
        out_type=(
            jax.ShapeDtypeStruct((bhr * s, 128), jnp.float32),
            jax.ShapeDtypeStruct((bhr, s // 128, 128), jnp.int32),
            jax.ShapeDtypeStruct((bhr, s // 128, 128), jnp.int32),
        ),
        mesh=mesh,
        scratch_types=[
            pltpu.VMEM((s,), jnp.int32),
            pltpu.VMEM((s,), jnp.int32),
            pltpu.VMEM((s // 128, 128), jnp.int32),
            pltpu.VMEM((s // 128, 128), jnp.int32),
            pltpu.VMEM((ch, 128), jnp.float32),
            pltpu.SemaphoreType.DMA,
        ],
        compiler_params=pltpu.CompilerParams(needs_layout_passes=False),
    )
    def body(posl_hbm, bkp_hbm, qkvf_hbm, sqkv_hbm, st_hbm, bkps_hbm,
             posl_v, bkp_v, st_v, bkps_v, bufqv, sem):
        wid = lax.axis_index("s") * SC_CORES + lax.axis_index("c")
        for t in range(tasks_per_w):
            task = wid * tasks_per_w + t
            bh = task // ROUND
            base_val = bh * s
            pltpu.sync_copy(posl_hbm.at[pl.ds(task * s, s)], posl_v)
            pltpu.sync_copy(bkp_hbm.at[pl.ds(bh * s, s)], bkp_v)

            def scat(i, _):
                idx = posl_v[pl.ds(i * 16, 16)]
                row = lax.shift_right_logical(idx, 7)
                lane = lax.bitwise_and(idx, 127)
                vals = lax.iota(jnp.int32, 16) + (i * 16 + base_val)
                plsc.store_scatter(st_v, [row, lane], vals)
                plsc.store_scatter(bkps_v, [row, lane],
                                   bkp_v[pl.ds(i * 16, 16)])
                return 0

            lax.fori_loop(0, s // 16, scat, 0, unroll=4)
            pltpu.sync_copy(st_v, st_hbm.at[task])
            pltpu.sync_copy(bkps_v, bkps_hbm.at[task])

            def gath(j, _):
                sl = st_v.at[j]
                pltpu.async_copy(qkvf_hbm.at[sl], bufqv, sem).wait()
                pltpu.sync_copy(
                    bufqv, sqkv_hbm.at[pl.ds(task * s + j * ch, ch)])
                return 0

            lax.fori_loop(0, nch, gath, 0)

    return body(posl, bkp, qkvf)


def _sc_unsort(posg, volf, *, bhr, s):
    tasks_per_w = bhr // SC_WORKERS
    ch = 128
    nch = s // ch
    mesh = plsc.VectorSubcoreMesh(core_axis_name="c", subcore_axis_name="s")

    @functools.partial(
        pl.kernel,
        out_type=jax.ShapeDtypeStruct((bhr * s, 128), jnp.float32),
        mesh=mesh,
        scratch_types=[
            pltpu.VMEM((s,), jnp.int32),
            pltpu.VMEM((ch, 128), jnp.float32),
            pltpu.SemaphoreType.DMA,
        ],
        compiler_params=pltpu.CompilerParams(needs_layout_passes=False),
    )
    def body(posg_hbm, volf_hbm, volu_hbm, idx_v, bufv, sem):
        wid = lax.axis_index("s") * SC_CORES + lax.axis_index("c")
        for t in range(tasks_per_w):
            task = wid * tasks_per_w + t
            pltpu.sync_copy(posg_hbm.at[pl.ds(task * s, s)], idx_v)

            def gath(j, _):
                sl = idx_v.at[pl.ds(j * ch, ch)]
                pltpu.async_copy(volf_hbm.at[sl], bufv, sem).wait()
                pltpu.sync_copy(
                    bufv, volu_hbm.at[pl.ds(task * s + j * ch, ch)])
                return 0

            lax.fori_loop(0, nch, gath, 0)

    return body(posg, volf)


def _win(a, lo, w, axis):
    n = a.shape[axis]
    lo = lo % n
    if lo + w <= n:
        return lax.slice_in_dim(a, lo, lo + w, axis=axis)
    first = lax.slice_in_dim(a, lo, n, axis=axis)
    second = lax.slice_in_dim(a, 0, lo + w - n, axis=axis)
    return jnp.concatenate([first, second], axis=axis)


_LOGDUP = tuple(float(np.log(k + 1e-9)) for k in range(ROUND + 1))


def _attn_body(sqkv_ref, mi_ref, mit_ref, volse_ref, *,
               s, cs, cb):
    qkv = sqkv_ref[0]
    qk = qkv[:, :KD]
    v = qkv[:, KD:]
    qm = mi_ref[0]
    kt = mit_ref[0]
    r_cur = pl.program_id(0) % ROUND
    rsh = r_cur * 6

    scale = KD ** -0.5
    nrm2 = jnp.dot(qk * qk, jnp.ones((KD, 1), jnp.float32),
                   preferred_element_type=jnp.float32)
    kscal = lax.rsqrt(jnp.maximum(nrm2, 1e-24)) * scale
    kn = qk * kscal

    rb = cb * cs
    w = rb + 2 * cs
    nblk = s // rb
    qi = lax.broadcasted_iota(jnp.int32, (rb, w), 0) // cs
    kj = lax.broadcasted_iota(jnp.int32, (rb, w), 1) // cs - 1
    invalid = jnp.abs(qi - kj) > 1

    for blk in range(nblk):
        lo = blk * rb - cs
        q = lax.slice_in_dim(qk, blk * rb, (blk + 1) * rb, axis=0)
        qmb = lax.slice_in_dim(qm, blk * rb, (blk + 1) * rb, axis=0)
        kwin = _win(kn, lo, w, 0)
        vwin = _win(v, lo, w, 0)
        ktw = _win(kt, lo, w, 1)

        dots = lax.dot_general(
            q, kwin, (((1,), (1,)), ((), ())),
            preferred_element_type=jnp.float32)
        qtick = qmb[:, 0:1]
        ktick = ktw[0:1, :]
        qp = qmb[:, 1:2]
        kp = ktw[1:2, :]

        x = lax.bitwise_xor(qp, kp)
        d = jnp.int32(0)
        for rr in range(ROUND):
            mrr = (lax.bitwise_and(x, 63 << (6 * rr)) == 0)
            d = d + mrr.astype(jnp.int32)
        logdup = jnp.where(
            d == 0, _LOGDUP[0],
            jnp.where(d == 1, _LOGDUP[1],
                      jnp.where(d == 2, _LOGDUP[2],
                                jnp.where(d == 3, _LOGDUP[3], _LOGDUP[4]))))
        qcur = lax.bitwise_and(lax.shift_right_logical(qp, rsh), 63)
        kcur = lax.bitwise_and(lax.shift_right_logical(kp, rsh), 63)
        killed = jnp.logical_or(qcur != kcur, invalid)

        dots = jnp.where(qtick == ktick, jnp.float32(-100000.0), dots)
        dots = jnp.where(killed, -NEG, dots) - logdup

        m = jnp.max(dots, axis=1, keepdims=True)
        e = jnp.exp(dots - m)
        ssum = jnp.sum(e, axis=1, keepdims=True)
        lse = m + jnp.log(ssum)
        p = e * (1.0 / ssum)
        vo = jnp.dot(p, vwin, preferred_element_type=jnp.float32)
        volse_ref[0, pl.ds(blk * rb, rb), :KD] = vo
        volse_ref[0, pl.ds(blk * rb, rb), KD:] = jnp.broadcast_to(
            lse, (rb, 128 - KD))


def k_attn(sqkv, meta, metat, cs, cb=8):
    bhr, s, _ = sqkv.shape
    return pl.pallas_call(
        functools.partial(_attn_body, s=s, cs=cs, cb=cb),
        grid=(bhr,),
        in_specs=[
            pl.BlockSpec((1, s, 128), lambda i: (i, 0, 0)),
            pl.BlockSpec((1, s, 8), lambda i: (i, 0, 0)),
            pl.BlockSpec((1, 8, s), lambda i: (i, 0, 0)),
        ],
        out_specs=pl.BlockSpec((1, s, 128), lambda i: (i, 0, 0)),
        out_shape=jax.ShapeDtypeStruct((bhr, s, 128), jnp.float32),
    )(sqkv, meta, metat)


def _combine_body(volu_ref, o_ref):
    ls = [volu_ref[0, rr][:, KD:KD + 1] for rr in range(ROUND)]
    m = ls[0]
    for rr in range(1, ROUND):
        m = jnp.maximum(m, ls[rr])
    ws = [jnp.exp(l - m) for l in ls]
    den = ws[0]
    for rr in range(1, ROUND):
        den = den + ws[rr]
    acc = volu_ref[0, 0][:, :KD] * ws[0]
    for rr in range(1, ROUND):
        acc = acc + volu_ref[0, rr][:, :KD] * ws[rr]
    o_ref[0] = acc / den


def k_combine(volu):
    bh, r, s, _ = volu.shape
    return pl.pallas_call(
        _combine_body,
        grid=(bh,),
        in_specs=[
            pl.BlockSpec((1, r, s, 128), lambda i: (i, 0, 0, 0)),
        ],
        out_specs=pl.BlockSpec((1, s, KD), lambda i: (i, 0, 0)),
        out_shape=jax.ShapeDtypeStruct((bh, s, KD), jnp.float32),
    )(volu)


def kernel(x, Wq, bq, Wv, bv, Wo, bo, hash_vec):
    b, s, d = x.shape
    bh = b * HEAD
    bhr = bh * ROUND
    cs = s // BUCKETS

    xf = x.reshape(b * s, d)
    wqv = jnp.concatenate([Wq.T, Wv.T], axis=1)
    bqv = jnp.concatenate([bq, bv], axis=0)
    proj = k_matmul(xf, wqv, bqv)
    qk = proj[:, :d].reshape(b, s, HEAD, KD).transpose(0, 2, 1, 3)
    v = proj[:, d:].reshape(b, s, HEAD, KD).transpose(0, 2, 1, 3)
    qk_h = qk.reshape(bh, s, KD)
    v_h = v.reshape(bh, s, KD)

    hv = hash_vec.transpose(0, 1, 3, 2).reshape(
        HEAD, KD, ROUND * (BUCKETS // 2))
    bq_ids = k_hash(qk_h, hv)

    bk = bq_ids.transpose(0, 2, 1).reshape(bhr, s // 128, 128)
    posl, posg = k_rank(bk)
    posl_f = posl.reshape(bhr * s)
    posg_f = posg.reshape(bhr * s)

    bkp = (bq_ids[..., 0] | (bq_ids[..., 1] << 6) | (bq_ids[..., 2] << 12)
           | (bq_ids[..., 3] << 18)).reshape(bh * s)

    qkv = jnp.concatenate(
        [qk_h.reshape(bh * s, KD), v_h.reshape(bh * s, KD)], axis=1)
    sqkv_f, st, bkps = _sc_sort_gather(posl_f, bkp, qkv, bhr=bhr, s=s)
    sqkv = sqkv_f.reshape(bhr, s, 128)

    tick = st.reshape(bhr, s) % s
    bkps2 = bkps.reshape(bhr, s)
    zpad = jnp.zeros((bhr, s), jnp.int32)
    meta = jnp.stack([tick, bkps2] + [zpad] * 6, axis=2)
    metat = meta.transpose(0, 2, 1)

    volse = k_attn(sqkv, meta, metat, cs)

    volu_f = _sc_unsort(posg_f, volse.reshape(bhr * s, 128), bhr=bhr, s=s)
    volu = volu_f.reshape(bh, ROUND, s, 128)

    ho = k_combine(volu)
    hflat = ho.reshape(b, HEAD, s, KD).transpose(0, 2, 1, 3).reshape(
        b * s, d)
    out = k_matmul(hflat, Wo.T, bo, bn=512)[: b * s].reshape(b, s, d)

    buckets_out = bq_ids.reshape(b, HEAD, s, ROUND).transpose(0, 1, 3, 2)
    return out, buckets_out


# speedup vs baseline: 1.0334x; 1.0334x over previous
import functools

import jax
import jax.numpy as jnp
import numpy as np
from jax import lax
from jax.experimental import pallas as pl
from jax.experimental.pallas import tpu as pltpu
from jax.experimental.pallas import tpu_sc as plsc

HEAD = 16
KD = 64
DM = 1024
ROUND = 4
BUCKETS = 64
NEG = float(np.finfo(np.float32).max)

SC_CORES = 2
SC_SUBCORES = 16
SC_WORKERS = SC_CORES * SC_SUBCORES


def _matmul_body(a_ref, w_ref, b_ref, o_ref):
    a = a_ref[...]
    w = w_ref[...]
    o_ref[...] = (
        jnp.dot(a, w, preferred_element_type=jnp.float32) + b_ref[...]
    )


def k_matmul(a, w, b, bm=512, bn=512):
    m, k = a.shape
    k2, n = w.shape
    assert k == k2 and b.shape == (n,)
    return pl.pallas_call(
        _matmul_body,
        grid=(m // bm, n // bn),
        in_specs=[
            pl.BlockSpec((bm, k), lambda i, j: (i, 0)),
            pl.BlockSpec((k, bn), lambda i, j: (0, j)),
            pl.BlockSpec((1, bn), lambda i, j: (0, j)),
        ],
        out_specs=pl.BlockSpec((bm, bn), lambda i, j: (i, j)),
        out_shape=jax.ShapeDtypeStruct((m, n), jnp.float32),
    )(a, w, b.reshape(1, n))


def _hash_body(qk_ref, hv_ref, o_ref):
    qk = qk_ref[0]
    hv = hv_ref[0]
    rot = jnp.dot(qk, hv, preferred_element_type=jnp.float32)
    s = rot.shape[0]
    cols = []
    half = BUCKETS // 2
    iota = lax.broadcasted_iota(jnp.int32, (s, BUCKETS), 1)
    for r in range(ROUND):
        xr = rot[:, r * half:(r + 1) * half]
        c = jnp.concatenate([-xr, xr], axis=1)
        m = jnp.max(c, axis=1, keepdims=True)
        idx = jnp.min(jnp.where(c == m, iota, BUCKETS), axis=1, keepdims=True)
        cols.append(idx)
    o_ref[0] = jnp.concatenate(cols, axis=1)


def k_hash(qk_h, hv):
    bh, s, _ = qk_h.shape
    return pl.pallas_call(
        _hash_body,
        grid=(bh,),
        in_specs=[
            pl.BlockSpec((1, s, KD), lambda i: (i, 0, 0)),
            pl.BlockSpec((1, KD, ROUND * (BUCKETS // 2)),
                         lambda i: (i % HEAD, 0, 0)),
        ],
        out_specs=pl.BlockSpec((1, s, ROUND), lambda i: (i, 0, 0)),
        out_shape=jax.ShapeDtypeStruct((bh, s, ROUND), jnp.int32),
    )(qk_h, hv)


def _rank_body(bk_ref, posl_ref, posg_ref, *, s):
    kk = bk_ref[0]
    sp = kk.shape[0]
    lane_lt = (lax.broadcasted_iota(jnp.int32, (128, 128), 0)
               < lax.broadcasted_iota(jnp.int32, (128, 128), 1)
               ).astype(jnp.float32)
    row_lt = (lax.broadcasted_iota(jnp.int32, (sp, sp), 0)
              > lax.broadcasted_iota(jnp.int32, (sp, sp), 1)
              ).astype(jnp.float32)
    pos = jnp.zeros((sp, 128), jnp.float32)
    acc = jnp.float32(0.0)
    for v in range(BUCKETS):
        eq = (kk == v).astype(jnp.float32)
        lane_prefix = jnp.dot(eq, lane_lt, preferred_element_type=jnp.float32)
        rowsum = jnp.sum(eq, axis=1, keepdims=True)
        rowsum_b = jnp.broadcast_to(rowsum, (sp, 128))
        rowexcl = jnp.dot(row_lt, rowsum_b,
                          preferred_element_type=jnp.float32)
        total = jnp.sum(eq)
        pos = pos + eq * (acc + rowexcl + lane_prefix)
        acc = acc + total
    posl = pos.astype(jnp.int32)
    posl_ref[0] = posl
    posg_ref[0] = posl + pl.program_id(0) * s


def k_rank(bk):
    bhr, sp, _ = bk.shape
    s = sp * 128
    return pl.pallas_call(
        functools.partial(_rank_body, s=s),
        grid=(bhr,),
        in_specs=[pl.BlockSpec((1, sp, 128), lambda i: (i, 0, 0))],
        out_specs=[
            pl.BlockSpec((1, sp, 128), lambda i: (i, 0, 0)),
            pl.BlockSpec((1, sp, 128), lambda i: (i, 0, 0)),
        ],
        out_shape=[
            jax.ShapeDtypeStruct((bhr, sp, 128), jnp.int32),
            jax.ShapeDtypeStruct((bhr, sp, 128), jnp.int32),
        ],
    )(bk)


def _sc_sort_gather(posl, bkp, qkvf, *, bhr, s):
    tasks_per_w = bhr // SC_WORKERS
    ch = 128
    nch = s // ch
    mesh = plsc.VectorSubcoreMesh(core_axis_name="c", subcore_axis_name="s")

    @functools.partial(
        pl.kernel,
        out_type=(
            jax.ShapeDtypeStruct((bhr * s, 128), jnp.float32),
            jax.ShapeDtypeStruct((bhr, s // 128, 128), jnp.int32),
            jax.ShapeDtypeStruct((bhr, s // 128, 128), jnp.int32),
        ),
        mesh=mesh,
        scratch_types=[
            pltpu.VMEM((s,), jnp.int32),
            pltpu.VMEM((s,), jnp.int32),
            pltpu.VMEM((s // 128, 128), jnp.int32),
            pltpu.VMEM((s // 128, 128), jnp.int32),
            pltpu.VMEM((ch, 128), jnp.float32),
            pltpu.SemaphoreType.DMA,
        ],
        compiler_params=pltpu.CompilerParams(needs_layout_passes=False),
    )
    def body(posl_hbm, bkp_hbm, qkvf_hbm, sqkv_hbm, st_hbm, bkps_hbm,
             posl_v, bkp_v, st_v, bkps_v, bufqv, sem):
        wid = lax.axis_index("s") * SC_CORES + lax.axis_index("c")
        for t in range(tasks_per_w):
            task = wid * tasks_per_w + t
            bh = task // ROUND
            base_val = bh * s
            pltpu.sync_copy(posl_hbm.at[pl.ds(task * s, s)], posl_v)
            pltpu.sync_copy(bkp_hbm.at[pl.ds(bh * s, s)], bkp_v)

            def scat(i, _):
                idx = posl_v[pl.ds(i * 16, 16)]
                row = lax.shift_right_logical(idx, 7)
                lane = lax.bitwise_and(idx, 127)
                vals = lax.iota(jnp.int32, 16) + (i * 16 + base_val)
                plsc.store_scatter(st_v, [row, lane], vals)
                plsc.store_scatter(bkps_v, [row, lane],
                                   bkp_v[pl.ds(i * 16, 16)])
                return 0

            lax.fori_loop(0, s // 16, scat, 0, unroll=4)
            pltpu.sync_copy(st_v, st_hbm.at[task])
            pltpu.sync_copy(bkps_v, bkps_hbm.at[task])

            def gath(j, _):
                sl = st_v.at[j]
                pltpu.async_copy(qkvf_hbm.at[sl], bufqv, sem).wait()
                pltpu.sync_copy(
                    bufqv, sqkv_hbm.at[pl.ds(task * s + j * ch, ch)])
                return 0

            lax.fori_loop(0, nch, gath, 0)

    return body(posl, bkp, qkvf)


def _sc_unsort(posg, volf, *, bhr, s):
    tasks_per_w = bhr // SC_WORKERS
    ch = 128
    nch = s // ch
    mesh = plsc.VectorSubcoreMesh(core_axis_name="c", subcore_axis_name="s")

    @functools.partial(
        pl.kernel,
        out_type=jax.ShapeDtypeStruct((bhr * s, 128), jnp.float32),
        mesh=mesh,
        scratch_types=[
            pltpu.VMEM((s,), jnp.int32),
            pltpu.VMEM((ch, 128), jnp.float32),
            pltpu.SemaphoreType.DMA,
        ],
        compiler_params=pltpu.CompilerParams(needs_layout_passes=False),
    )
    def body(posg_hbm, volf_hbm, volu_hbm, idx_v, bufv, sem):
        wid = lax.axis_index("s") * SC_CORES + lax.axis_index("c")
        for t in range(tasks_per_w):
            task = wid * tasks_per_w + t
            pltpu.sync_copy(posg_hbm.at[pl.ds(task * s, s)], idx_v)

            def gath(j, _):
                sl = idx_v.at[pl.ds(j * ch, ch)]
                pltpu.async_copy(volf_hbm.at[sl], bufv, sem).wait()
                pltpu.sync_copy(
                    bufv, volu_hbm.at[pl.ds(task * s + j * ch, ch)])
                return 0

            lax.fori_loop(0, nch, gath, 0)

    return body(posg, volf)


def _win(a, lo, w, axis):
    n = a.shape[axis]
    lo = lo % n
    if lo + w <= n:
        return lax.slice_in_dim(a, lo, lo + w, axis=axis)
    first = lax.slice_in_dim(a, lo, n, axis=axis)
    second = lax.slice_in_dim(a, 0, lo + w - n, axis=axis)
    return jnp.concatenate([first, second], axis=axis)


_LOGDUP = tuple(float(np.log(k + 1e-9)) for k in range(ROUND + 1))


def _attn_body(sqkv_ref, mi_ref, mit_ref, volse_ref, *,
               s, cs, cb):
    qkv = sqkv_ref[0]
    qk = qkv[:, :KD]
    v = qkv[:, KD:]
    qm = mi_ref[0]
    kt = mit_ref[0]
    r_cur = pl.program_id(0) % ROUND
    rsh = r_cur * 6

    scale = KD ** -0.5
    nrm2 = jnp.dot(qk * qk, jnp.ones((KD, 1), jnp.float32),
                   preferred_element_type=jnp.float32)
    kscal = lax.rsqrt(jnp.maximum(nrm2, 1e-24)) * scale
    kn = qk * kscal

    rb = cb * cs
    w = rb + 2 * cs
    nblk = s // rb
    qi = lax.broadcasted_iota(jnp.int32, (rb, w), 0) // cs
    kj = lax.broadcasted_iota(jnp.int32, (rb, w), 1) // cs - 1
    invalid = jnp.abs(qi - kj) > 1

    for blk in range(nblk):
        lo = blk * rb - cs
        q = lax.slice_in_dim(qk, blk * rb, (blk + 1) * rb, axis=0)
        qmb = lax.slice_in_dim(qm, blk * rb, (blk + 1) * rb, axis=0)
        kwin = _win(kn, lo, w, 0)
        vwin = _win(v, lo, w, 0)
        ktw = _win(kt, lo, w, 1)

        dots = lax.dot_general(
            q, kwin, (((1,), (1,)), ((), ())),
            preferred_element_type=jnp.float32)
        qtick = qmb[:, 0:1]
        ktick = ktw[0:1, :]
        qp = qmb[:, 1:2]
        kp = ktw[1:2, :]

        x = lax.bitwise_xor(qp, kp)
        d = jnp.int32(0)
        for rr in range(ROUND):
            mrr = (lax.bitwise_and(x, 63 << (6 * rr)) == 0)
            d = d + mrr.astype(jnp.int32)
        logdup = jnp.where(
            d == 0, _LOGDUP[0],
            jnp.where(d == 1, _LOGDUP[1],
                      jnp.where(d == 2, _LOGDUP[2],
                                jnp.where(d == 3, _LOGDUP[3], _LOGDUP[4]))))
        qcur = lax.bitwise_and(lax.shift_right_logical(qp, rsh), 63)
        kcur = lax.bitwise_and(lax.shift_right_logical(kp, rsh), 63)
        killed = jnp.logical_or(qcur != kcur, invalid)

        dots = jnp.where(qtick == ktick, jnp.float32(-100000.0), dots)
        dots = jnp.where(killed, -NEG, dots) - logdup

        m = jnp.max(dots, axis=1, keepdims=True)
        e = jnp.exp(dots - m)
        ssum = jnp.sum(e, axis=1, keepdims=True)
        lse = m + jnp.log(ssum)
        p = e * (1.0 / ssum)
        vo = jnp.dot(p, vwin, preferred_element_type=jnp.float32)
        volse_ref[0, pl.ds(blk * rb, rb), :KD] = vo
        volse_ref[0, pl.ds(blk * rb, rb), KD:] = jnp.broadcast_to(
            lse, (rb, 128 - KD))


def k_attn(sqkv, meta, metat, cs, cb=8):
    bhr, s, _ = sqkv.shape
    return pl.pallas_call(
        functools.partial(_attn_body, s=s, cs=cs, cb=cb),
        grid=(bhr,),
        in_specs=[
            pl.BlockSpec((1, s, 128), lambda i: (i, 0, 0)),
            pl.BlockSpec((1, s, 8), lambda i: (i, 0, 0)),
            pl.BlockSpec((1, 8, s), lambda i: (i, 0, 0)),
        ],
        out_specs=pl.BlockSpec((1, s, 128), lambda i: (i, 0, 0)),
        out_shape=jax.ShapeDtypeStruct((bhr, s, 128), jnp.float32),
    )(sqkv, meta, metat)


def _combine_body(volu_ref, o_ref):
    ls = [volu_ref[0, rr][:, KD:KD + 1] for rr in range(ROUND)]
    m = ls[0]
    for rr in range(1, ROUND):
        m = jnp.maximum(m, ls[rr])
    ws = [jnp.exp(l - m) for l in ls]
    den = ws[0]
    for rr in range(1, ROUND):
        den = den + ws[rr]
    acc = volu_ref[0, 0][:, :KD] * ws[0]
    for rr in range(1, ROUND):
        acc = acc + volu_ref[0, rr][:, :KD] * ws[rr]
    o_ref[0] = acc / den


def k_combine(volu):
    bh, r, s, _ = volu.shape
    return pl.pallas_call(
        _combine_body,
        grid=(bh,),
        in_specs=[
            pl.BlockSpec((1, r, s, 128), lambda i: (i, 0, 0, 0)),
        ],
        out_specs=pl.BlockSpec((1, s, KD), lambda i: (i, 0, 0)),
        out_shape=jax.ShapeDtypeStruct((bh, s, KD), jnp.float32),
    )(volu)


def kernel(x, Wq, bq, Wv, bv, Wo, bo, hash_vec):
    b, s, d = x.shape
    bh = b * HEAD
    bhr = bh * ROUND
    cs = s // BUCKETS

    xf = x.reshape(b * s, d)
    wqv = jnp.concatenate([Wq.T, Wv.T], axis=1)
    bqv = jnp.concatenate([bq, bv], axis=0)
    proj = k_matmul(xf, wqv, bqv)
    qk = proj[:, :d].reshape(b, s, HEAD, KD).transpose(0, 2, 1, 3)
    v = proj[:, d:].reshape(b, s, HEAD, KD).transpose(0, 2, 1, 3)
    qk_h = qk.reshape(bh, s, KD)
    v_h = v.reshape(bh, s, KD)

    hv = hash_vec.transpose(0, 1, 3, 2).reshape(
        HEAD, KD, ROUND * (BUCKETS // 2))
    bq_ids = k_hash(qk_h, hv)

    bk = bq_ids.transpose(0, 2, 1).reshape(bhr, s // 128, 128)
    posl, posg = k_rank(bk)
    posl_f = posl.reshape(bhr * s)
    posg_f = posg.reshape(bhr * s)

    bkp = (bq_ids[..., 0] | (bq_ids[..., 1] << 6) | (bq_ids[..., 2] << 12)
           | (bq_ids[..., 3] << 18)).reshape(bh * s)

    qkv = jnp.concatenate(
        [qk_h.reshape(bh * s, KD), v_h.reshape(bh * s, KD)], axis=1)
    sqkv_f, st, bkps = _sc_sort_gather(posl_f, bkp, qkv, bhr=bhr, s=s)
    sqkv = sqkv_f.reshape(bhr, s, 128)

    tick = st.reshape(bhr, s) % s
    bkps2 = bkps.reshape(bhr, s)
    zpad = jnp.zeros((bhr, s), jnp.int32)
    meta = jnp.stack([tick, bkps2] + [zpad] * 6, axis=2)
    metat = meta.transpose(0, 2, 1)

    volse = k_attn(sqkv, meta, metat, cs)

    volu_f = _sc_unsort(posg_f, volse.reshape(bhr * s, 128), bhr=bhr, s=s)
    volu = volu_f.reshape(bh, ROUND, s, 128)

    ho = k_combine(volu)
    hflat = ho.reshape(b, HEAD, s, KD).transpose(0, 2, 1, 3).reshape(
        b * s, d)
    out = k_matmul(hflat, Wo.T, bo, bn=512)[: b * s].reshape(b, s, d)

    buckets_out = bq_ids.reshape(b, HEAD, s, ROUND).transpose(0, 1, 3, 2)
    return out, buckets_out
